# Initial kernel scaffold; baseline (speedup 1.0000x reference)
import jax
import jax.numpy as jnp
from jax.experimental import pallas as pl


def kernel(feature, text_feature, centers, text_centers, position, mem_feats, vid, domains):
    raise NotImplementedError("write your pallas kernel here")


# speedup vs baseline: 2.3217x; 2.3217x over previous
import functools

import jax
import jax.numpy as jnp
from jax import lax
from jax.experimental import pallas as pl
from jax.experimental.pallas import tpu as pltpu
from jax.experimental.pallas import tpu_sc as plsc

SCALE = 10.0
K = 10
TEMP = 0.5
SIM_THR = 0.7
KLW = max(0.5 * (1 - 1 / 60.0), 0.1)

P, B, D, M = 4, 1024, 256, 16384
NC, NS = 2, 16
NW = NC * NS
GCHUNK = 128
ROWS_PER_W = (B * K * P) // NW
MBLK = 2048


def _l2n(x):
    n = jnp.sqrt(jnp.sum(x * x, axis=1, keepdims=True))
    return x / jnp.maximum(n, 1e-12)


def _dotT(a, b):
    return lax.dot_general(a, b, dimension_numbers=(((1,), (1,)), ((), ())),
                           preferred_element_type=jnp.float32)


def _align_body(f_ref, t_ref, out_ref):
    p = pl.program_id(0)
    f = f_ref[0]
    t = t_ref[0]
    n1 = _l2n(f)
    t1 = _l2n(t)
    img_sim = _dotT(n1, n1) * (1.0 / TEMP)
    txt_sim = _dotT(t1, t1) * (1.0 / TEMP)
    mask = (img_sim > SIM_THR) & (txt_sim > SIM_THR)
    valid = jnp.any(mask, axis=1)
    vf = valid.astype(jnp.float32)
    col = jnp.where(valid[None, :], 0.0, -1e9)
    img_s = img_sim + col
    txt_s = txt_sim + col

    def lse(s):
        m = jnp.max(s, axis=-1, keepdims=True)
        return m + jnp.log(jnp.sum(jnp.exp(s - m), axis=-1, keepdims=True))

    img_lp = img_s - lse(img_s)
    txt_lp = txt_s - lse(txt_s)
    t_p = jnp.exp(txt_lp)
    i_p = jnp.exp(img_lp)
    w = vf[:, None] * vf[None, :]
    n = jnp.maximum(jnp.sum(vf), 1.0)
    kl1 = jnp.sum(w * t_p * (txt_lp - img_lp)) / n
    kl2 = jnp.sum(w * i_p * (img_lp - txt_lp)) / n
    lk = 0.5 * (kl1 + kl2)
    lk = jnp.where(jnp.sum(vf) > 0, lk, 0.0)
    prev = jnp.where(p == 0, 0.0, out_ref[0, 0])
    out_ref[...] = jnp.reshape(prev + lk, (1, 1))


def _align_call(feature, tft):
    return pl.pallas_call(
        _align_body,
        grid=(P,),
        in_specs=[
            pl.BlockSpec((1, B, D), lambda p: (p, 0, 0)),
            pl.BlockSpec((1, B, D), lambda p: (p, 0, 0)),
        ],
        out_specs=pl.BlockSpec((1, 1), lambda p: (0, 0)),
        out_shape=jax.ShapeDtypeStruct((1, 1), jnp.float32),
    )(feature, tft)


RB = 256


def _topk_body(t_ref, m_ref, out_ref):
    tf = t_ref[...]
    q = _l2n(jnp.mean(tf, axis=0))
    mn = _l2n(m_ref[...])
    s = _dotT(q, mn)
    iota = lax.broadcasted_iota(jnp.int32, (RB, M), 1)
    cols = []
    for _t in range(K):
        cur = jnp.max(s, axis=1)
        am = jnp.min(jnp.where(s == cur[:, None], iota, M), axis=1)
        cols.append(am)
        s = jnp.where(iota == am[:, None], -jnp.inf, s)
    out_ref[...] = jnp.stack(cols, axis=1)


def _topk_call(tft, mem_feats):
    return pl.pallas_call(
        _topk_body,
        grid=(B // RB,),
        in_specs=[
            pl.BlockSpec((P, RB, D), lambda i: (0, i, 0)),
            pl.BlockSpec((M, D), lambda i: (0, 0)),
        ],
        out_specs=pl.BlockSpec((RB, K), lambda i: (i, 0)),
        out_shape=jax.ShapeDtypeStruct((B, K), jnp.int32),
    )(tft, mem_feats)


def _sc_body(cflat, cross, posn, vid, posf, negm, pvout,
             idxr, idx2, rows, mask_v, vid_v, pos_v, outb, sem):
    wid = lax.axis_index("s") * NC + lax.axis_index("c")
    pp = wid // (NW // P)
    wb = (wid % (NW // P)) * ROWS_PER_W

    def chunk(j, _):
        gbase = wb + j * GCHUNK
        pltpu.sync_copy(cross.at[pl.ds(gbase, GCHUNK)], idxr)

        def addp(i, _):
            idx2[pl.ds(i * 16, 16)] = idxr[pl.ds(i * 16, 16)] + pp * M
            return 0
        lax.fori_loop(0, GCHUNK // 16, addp, 0)
        pltpu.async_copy(cflat.at[idx2], rows, sem).wait()
        pltpu.sync_copy(rows, posf.at[pl.ds(pp * (B * K) + gbase, GCHUNK)])
        return 0
    lax.fori_loop(0, ROWS_PER_W // GCHUNK, chunk, 0)

    zeros16 = jnp.zeros((16,), jnp.float32)

    @pl.when(wid == 0)
    def _():
        def init(i, _):
            mask_v[pl.ds(i * 16, 16)] = jnp.ones((16,), jnp.float32)
            return 0
        lax.fori_loop(0, M // 16, init, 0)

        def cscat(cidx, _):
            pltpu.sync_copy(cross.at[pl.ds(cidx * GCHUNK, GCHUNK)], idxr)

            def inner(j, _):
                iv = idxr[pl.ds(j * 16, 16)]
                plsc.store_scatter(mask_v, [iv], zeros16)
                return 0
            lax.fori_loop(0, GCHUNK // 16, inner, 0)
            return 0
        lax.fori_loop(0, (B * K) // GCHUNK, cscat, 0)
        pltpu.sync_copy(posn, pos_v)

        def pscat(i, _):
            iv = pos_v[pl.ds(i * 16, 16)]
            plsc.store_scatter(mask_v, [iv], zeros16)
            return 0
        lax.fori_loop(0, B // 16, pscat, 0)
        pltpu.sync_copy(mask_v, negm)

    @pl.when(wid == 1)
    def _():
        pltpu.sync_copy(vid, vid_v)

        def vchunk(cidx, _):
            pltpu.sync_copy(cross.at[pl.ds(cidx * GCHUNK, GCHUNK)], idxr)

            def inner(j, _):
                iv = idxr[pl.ds(j * 16, 16)]
                outb[pl.ds(j * 16, 16)] = plsc.load_gather(vid_v, [iv])
                return 0
            lax.fori_loop(0, GCHUNK // 16, inner, 0)
            pltpu.sync_copy(outb, pvout.at[pl.ds(cidx * GCHUNK, GCHUNK)])
            return 0
        lax.fori_loop(0, (B * K) // GCHUNK, vchunk, 0)


@functools.lru_cache(maxsize=1)
def _make_sc_call():
    return functools.partial(
        pl.kernel,
        out_type=[
            jax.ShapeDtypeStruct((P * B * K, D), jnp.float32),
            jax.ShapeDtypeStruct((M,), jnp.float32),
            jax.ShapeDtypeStruct((B * K,), jnp.int32),
        ],
        mesh=plsc.VectorSubcoreMesh(core_axis_name="c", subcore_axis_name="s",
                                    num_cores=NC, num_subcores=NS),
        compiler_params=pltpu.CompilerParams(needs_layout_passes=False),
        scratch_types=[
            pltpu.VMEM((GCHUNK,), jnp.int32),
            pltpu.VMEM((GCHUNK,), jnp.int32),
            pltpu.VMEM((GCHUNK, D), jnp.float32),
            pltpu.VMEM((M,), jnp.float32),
            pltpu.VMEM((M,), jnp.int32),
            pltpu.VMEM((B,), jnp.int32),
            pltpu.VMEM((GCHUNK,), jnp.int32),
            pltpu.SemaphoreType.DMA,
        ],
    )(_sc_body)


def _sc_call(cflat, crossf, posn, vid):
    return _make_sc_call()(cflat, crossf, posn, vid)


def _y_body(f_ref, c_ref, nm_ref, out_ref):
    mb = pl.program_id(1)
    f = f_ref[0]
    c = c_ref[0]
    nm = nm_ref[0]
    a2 = jnp.sum(f * f, axis=1)
    b2 = jnp.sum(c * c, axis=1)
    g = _dotT(f, c)
    d2 = jnp.maximum(a2[:, None] + b2[None, :] - 2.0 * g, 0.0)
    e = jnp.exp(-SCALE * jnp.sqrt(d2 + 1e-12))
    part = jnp.sum(e * nm[None, :], axis=1)
    prev = jnp.where(mb == 0, jnp.zeros_like(part), out_ref[0, 0])
    out_ref[...] = (prev + part)[None, None, :]


def _y_call(feature, centers, negm2d):
    return pl.pallas_call(
        _y_body,
        grid=(P, M // MBLK),
        in_specs=[
            pl.BlockSpec((1, B, D), lambda p, mb: (p, 0, 0)),
            pl.BlockSpec((1, MBLK, D), lambda p, mb: (p, mb, 0)),
            pl.BlockSpec((1, MBLK), lambda p, mb: (0, mb)),
        ],
        out_specs=pl.BlockSpec((1, 1, B), lambda p, mb: (p, 0, 0)),
        out_shape=jax.ShapeDtypeStruct((P, 1, B), jnp.float32),
    )(feature, centers, negm2d)


def _final_body(pf_ref, f_ref, ys_ref, al_ref, tot_ref, con_ref, acc_ref):
    p = pl.program_id(0)
    pf = pf_ref[0].reshape(B, K, D)
    f = f_ref[0]
    diff = f[:, None, :] - pf
    d2 = jnp.sum(diff * diff, axis=-1) + 1e-12
    x = jnp.log(jnp.sum(jnp.exp(-SCALE * jnp.sqrt(d2)), axis=1))
    y = jnp.log(ys_ref[0, 0])
    l = jnp.sum(y - x) / B
    l = jnp.where(jnp.isnan(l), 0.0, l)
    prev = jnp.where(p == 0, 0.0, acc_ref[0, 0])
    tot = prev + l
    acc_ref[...] = jnp.reshape(tot, (1, 1))

    @pl.when(p == P - 1)
    def _():
        con = tot / P
        con_ref[...] = jnp.reshape(con, (1, 1))
        tot_ref[...] = jnp.reshape(con + KLW * al_ref[0, 0], (1, 1))


def _final_call(posf3, feature, ysum, align):
    return pl.pallas_call(
        _final_body,
        grid=(P,),
        in_specs=[
            pl.BlockSpec((1, B * K, D), lambda p: (p, 0, 0)),
            pl.BlockSpec((1, B, D), lambda p: (p, 0, 0)),
            pl.BlockSpec((1, 1, B), lambda p: (p, 0, 0)),
            pl.BlockSpec((1, 1), lambda p: (0, 0)),
        ],
        out_specs=[
            pl.BlockSpec((1, 1), lambda p: (0, 0)),
            pl.BlockSpec((1, 1), lambda p: (0, 0)),
        ],
        out_shape=[
            jax.ShapeDtypeStruct((1, 1), jnp.float32),
            jax.ShapeDtypeStruct((1, 1), jnp.float32),
        ],
        scratch_shapes=[pltpu.VMEM((1, 1), jnp.float32)],
    )(posf3, feature, ysum, align)


def kernel(feature, text_feature, centers, text_centers, position,
           mem_feats, vid, domains):
    tft = jnp.transpose(text_feature, (1, 0, 2))
    align = _align_call(feature, tft)
    cross = _topk_call(tft, mem_feats)
    posf, negm, pvf = _sc_call(
        centers.reshape(P * M, D), cross.reshape(-1),
        position.astype(jnp.int32), vid)
    ysum = _y_call(feature, centers, negm.reshape(1, M))
    total, contr = _final_call(posf.reshape(P, B * K, D), feature, ysum, align)
    return (total.reshape(()), contr.reshape(()), align.reshape(()),
            pvf.reshape(B, K))


# speedup vs baseline: 2.5241x; 1.0872x over previous
import functools

import jax
import jax.numpy as jnp
from jax import lax
from jax.experimental import pallas as pl
from jax.experimental.pallas import tpu as pltpu
from jax.experimental.pallas import tpu_sc as plsc

SCALE = 10.0
K = 10
TEMP = 0.5
SIM_THR = 0.7
KLW = max(0.5 * (1 - 1 / 60.0), 0.1)

P, B, D, M = 4, 1024, 256, 16384
NC, NS = 2, 16
NW = NC * NS
GCHUNK = 128
ROWS_PER_W = (B * K * P) // NW
MBLK = 2048


def _l2n(x):
    n = jnp.sqrt(jnp.sum(x * x, axis=1, keepdims=True))
    return x / jnp.maximum(n, 1e-12)


def _dotT(a, b):
    return lax.dot_general(a, b, dimension_numbers=(((1,), (1,)), ((), ())),
                           preferred_element_type=jnp.float32)


def _align_body(f_ref, t_ref, out_ref):
    p = pl.program_id(0)
    f = f_ref[0]
    t = t_ref[0]
    n1 = _l2n(f)
    t1 = _l2n(t)
    img_sim = _dotT(n1, n1) * (1.0 / TEMP)
    txt_sim = _dotT(t1, t1) * (1.0 / TEMP)
    mask = (img_sim > SIM_THR) & (txt_sim > SIM_THR)
    valid = jnp.any(mask, axis=1)
    vf = valid.astype(jnp.float32)
    col = jnp.where(valid[None, :], 0.0, -1e9)
    img_s = img_sim + col
    txt_s = txt_sim + col

    def lse(s):
        m = jnp.max(s, axis=-1, keepdims=True)
        return m + jnp.log(jnp.sum(jnp.exp(s - m), axis=-1, keepdims=True))

    img_lp = img_s - lse(img_s)
    txt_lp = txt_s - lse(txt_s)
    t_p = jnp.exp(txt_lp)
    i_p = jnp.exp(img_lp)
    w = vf[:, None] * vf[None, :]
    n = jnp.maximum(jnp.sum(vf), 1.0)
    kl1 = jnp.sum(w * t_p * (txt_lp - img_lp)) / n
    kl2 = jnp.sum(w * i_p * (img_lp - txt_lp)) / n
    lk = 0.5 * (kl1 + kl2)
    lk = jnp.where(jnp.sum(vf) > 0, lk, 0.0)
    prev = jnp.where(p == 0, 0.0, out_ref[0, 0])
    out_ref[...] = jnp.reshape(prev + lk, (1, 1))


def _align_call(feature, tft):
    return pl.pallas_call(
        _align_body,
        grid=(P,),
        in_specs=[
            pl.BlockSpec((1, B, D), lambda p: (p, 0, 0)),
            pl.BlockSpec((1, B, D), lambda p: (p, 0, 0)),
        ],
        out_specs=pl.BlockSpec((1, 1), lambda p: (0, 0)),
        out_shape=jax.ShapeDtypeStruct((1, 1), jnp.float32),
    )(feature, tft)


RB = 256


def _topk_body(t_ref, m_ref, out_ref):
    tf = t_ref[...]
    q = _l2n(jnp.mean(tf, axis=0))
    mn = _l2n(m_ref[...])
    s = _dotT(q, mn)
    iota = lax.broadcasted_iota(jnp.int32, (RB, M), 1)
    cols = []
    for _t in range(K):
        cur = jnp.max(s, axis=1)
        am = jnp.min(jnp.where(s == cur[:, None], iota, M), axis=1)
        cols.append(am)
        s = jnp.where(iota == am[:, None], -jnp.inf, s)
    out_ref[...] = jnp.stack(cols, axis=1)


def _topk_call(tft, mem_feats):
    return pl.pallas_call(
        _topk_body,
        grid=(B // RB,),
        in_specs=[
            pl.BlockSpec((P, RB, D), lambda i: (0, i, 0)),
            pl.BlockSpec((M, D), lambda i: (0, 0)),
        ],
        out_specs=pl.BlockSpec((RB, K), lambda i: (i, 0)),
        out_shape=jax.ShapeDtypeStruct((B, K), jnp.int32),
    )(tft, mem_feats)


def _sc_body(cflat, cross, posn, vid, posf, negm, pvout,
             idxsl, idx2, rows0, rows1, cfull, pvbuf, mask_v, vid_v, pos_v,
             gsem0, gsem1, osem0, osem1):
    wid = lax.axis_index("s") * NC + lax.axis_index("c")
    pp = wid // (NW // P)
    wb = (wid % (NW // P)) * ROWS_PER_W

    zeros16 = jnp.zeros((16,), jnp.float32)

    @pl.when(wid == 0)
    def _():
        pltpu.sync_copy(cross, cfull)
        pltpu.sync_copy(posn, pos_v)

        def init(i, _):
            mask_v[pl.ds(i * 16, 16)] = jnp.ones((16,), jnp.float32)
            return 0
        lax.fori_loop(0, M // 16, init, 0)

        def cscat(j, _):
            plsc.store_scatter(mask_v, [cfull[pl.ds(j * 16, 16)]], zeros16)
            return 0
        lax.fori_loop(0, (B * K) // 16, cscat, 0)

        def pscat(i, _):
            plsc.store_scatter(mask_v, [pos_v[pl.ds(i * 16, 16)]], zeros16)
            return 0
        lax.fori_loop(0, B // 16, pscat, 0)
        pltpu.sync_copy(mask_v, negm)

    @pl.when(wid == 1)
    def _():
        pltpu.sync_copy(cross, cfull)
        pltpu.sync_copy(vid, vid_v)

        def vg(j, _):
            pvbuf[pl.ds(j * 16, 16)] = plsc.load_gather(
                vid_v, [cfull[pl.ds(j * 16, 16)]])
            return 0
        lax.fori_loop(0, (B * K) // 16, vg, 0)
        pltpu.sync_copy(pvbuf, pvout)

    pltpu.sync_copy(cross.at[pl.ds(wb, ROWS_PER_W)], idxsl)

    def addp(i, _):
        idx2[pl.ds(i * 16, 16)] = idxsl[pl.ds(i * 16, 16)] + pp * M
        return 0
    lax.fori_loop(0, ROWS_PER_W // 16, addp, 0)

    nchunk = ROWS_PER_W // GCHUNK
    rows = (rows0, rows1)
    gsem = (gsem0, gsem1)
    osem = (osem0, osem1)
    obase = pp * (B * K) + wb
    for j in range(2):
        pltpu.async_copy(cflat.at[idx2.at[pl.ds(j * GCHUNK, GCHUNK)]],
                         rows[j], gsem[j])
    for j in range(nchunk):
        b = j % 2
        pltpu.make_async_copy(cflat.at[idx2.at[pl.ds(0, GCHUNK)]],
                              rows[b], gsem[b]).wait()
        pltpu.async_copy(rows[b], posf.at[pl.ds(obase + j * GCHUNK, GCHUNK)],
                         osem[b])
        if j + 2 < nchunk:
            pltpu.make_async_copy(rows[b],
                                  posf.at[pl.ds(obase, GCHUNK)],
                                  osem[b]).wait()
            pltpu.async_copy(
                cflat.at[idx2.at[pl.ds((j + 2) * GCHUNK, GCHUNK)]],
                rows[b], gsem[b])
    for j in range(2):
        pltpu.make_async_copy(rows[j], posf.at[pl.ds(obase, GCHUNK)],
                              osem[j]).wait()


@functools.lru_cache(maxsize=1)
def _make_sc_call():
    return functools.partial(
        pl.kernel,
        out_type=[
            jax.ShapeDtypeStruct((P * B * K, D), jnp.float32),
            jax.ShapeDtypeStruct((M,), jnp.float32),
            jax.ShapeDtypeStruct((B * K,), jnp.int32),
        ],
        mesh=plsc.VectorSubcoreMesh(core_axis_name="c", subcore_axis_name="s",
                                    num_cores=NC, num_subcores=NS),
        compiler_params=pltpu.CompilerParams(needs_layout_passes=False),
        scratch_types=[
            pltpu.VMEM((ROWS_PER_W,), jnp.int32),
            pltpu.VMEM((ROWS_PER_W,), jnp.int32),
            pltpu.VMEM((GCHUNK, D), jnp.float32),
            pltpu.VMEM((GCHUNK, D), jnp.float32),
            pltpu.VMEM((B * K,), jnp.int32),
            pltpu.VMEM((B * K,), jnp.int32),
            pltpu.VMEM((M,), jnp.float32),
            pltpu.VMEM((M,), jnp.int32),
            pltpu.VMEM((B,), jnp.int32),
            pltpu.SemaphoreType.DMA,
            pltpu.SemaphoreType.DMA,
            pltpu.SemaphoreType.DMA,
            pltpu.SemaphoreType.DMA,
        ],
    )(_sc_body)


def _sc_call(cflat, crossf, posn, vid):
    return _make_sc_call()(cflat, crossf, posn, vid)


def _y_body(f_ref, c_ref, nm_ref, out_ref):
    mb = pl.program_id(1)
    f = f_ref[0]
    c = c_ref[0]
    nm = nm_ref[0]
    a2 = jnp.sum(f * f, axis=1)
    b2 = jnp.sum(c * c, axis=1)
    g = lax.dot_general(f.astype(jnp.bfloat16), c.astype(jnp.bfloat16),
                        dimension_numbers=(((1,), (1,)), ((), ())),
                        preferred_element_type=jnp.float32)
    d2 = jnp.maximum(a2[:, None] + b2[None, :] - 2.0 * g, 0.0)
    e = jnp.exp(-SCALE * jnp.sqrt(d2 + 1e-12))
    part = jnp.sum(e * nm[None, :], axis=1)
    prev = jnp.where(mb == 0, jnp.zeros_like(part), out_ref[0, 0])
    out_ref[...] = (prev + part)[None, None, :]


def _y_call(feature, centers, negm2d):
    return pl.pallas_call(
        _y_body,
        grid=(P, M // MBLK),
        in_specs=[
            pl.BlockSpec((1, B, D), lambda p, mb: (p, 0, 0)),
            pl.BlockSpec((1, MBLK, D), lambda p, mb: (p, mb, 0)),
            pl.BlockSpec((1, MBLK), lambda p, mb: (0, mb)),
        ],
        out_specs=pl.BlockSpec((1, 1, B), lambda p, mb: (p, 0, 0)),
        out_shape=jax.ShapeDtypeStruct((P, 1, B), jnp.float32),
    )(feature, centers, negm2d)


def _final_body(pf_ref, f_ref, ys_ref, al_ref, tot_ref, con_ref, acc_ref):
    p = pl.program_id(0)
    pf = pf_ref[0].reshape(B, K, D)
    f = f_ref[0]
    diff = f[:, None, :] - pf
    d2 = jnp.sum(diff * diff, axis=-1) + 1e-12
    x = jnp.log(jnp.sum(jnp.exp(-SCALE * jnp.sqrt(d2)), axis=1))
    y = jnp.log(ys_ref[0, 0])
    l = jnp.sum(y - x) / B
    l = jnp.where(jnp.isnan(l), 0.0, l)
    prev = jnp.where(p == 0, 0.0, acc_ref[0, 0])
    tot = prev + l
    acc_ref[...] = jnp.reshape(tot, (1, 1))

    @pl.when(p == P - 1)
    def _():
        con = tot / P
        con_ref[...] = jnp.reshape(con, (1, 1))
        tot_ref[...] = jnp.reshape(con + KLW * al_ref[0, 0], (1, 1))


def _final_call(posf3, feature, ysum, align):
    return pl.pallas_call(
        _final_body,
        grid=(P,),
        in_specs=[
            pl.BlockSpec((1, B * K, D), lambda p: (p, 0, 0)),
            pl.BlockSpec((1, B, D), lambda p: (p, 0, 0)),
            pl.BlockSpec((1, 1, B), lambda p: (p, 0, 0)),
            pl.BlockSpec((1, 1), lambda p: (0, 0)),
        ],
        out_specs=[
            pl.BlockSpec((1, 1), lambda p: (0, 0)),
            pl.BlockSpec((1, 1), lambda p: (0, 0)),
        ],
        out_shape=[
            jax.ShapeDtypeStruct((1, 1), jnp.float32),
            jax.ShapeDtypeStruct((1, 1), jnp.float32),
        ],
        scratch_shapes=[pltpu.VMEM((1, 1), jnp.float32)],
    )(posf3, feature, ysum, align)


def kernel(feature, text_feature, centers, text_centers, position,
           mem_feats, vid, domains):
    tft = jnp.transpose(text_feature, (1, 0, 2))
    align = _align_call(feature, tft)
    cross = _topk_call(tft, mem_feats)
    posf, negm, pvf = _sc_call(
        centers.reshape(P * M, D), cross.reshape(-1),
        position.astype(jnp.int32), vid)
    ysum = _y_call(feature, centers, negm.reshape(1, M))
    total, contr = _final_call(posf.reshape(P, B * K, D), feature, ysum, align)
    return (total.reshape(()), contr.reshape(()), align.reshape(()),
            pvf.reshape(B, K))


# speedup vs baseline: 2.5945x; 1.0279x over previous
import functools

import jax
import jax.numpy as jnp
from jax import lax
from jax.experimental import pallas as pl
from jax.experimental.pallas import tpu as pltpu
from jax.experimental.pallas import tpu_sc as plsc

SCALE = 10.0
K = 10
TEMP = 0.5
SIM_THR = 0.7
KLW = max(0.5 * (1 - 1 / 60.0), 0.1)

P, B, D, M = 4, 1024, 256, 16384
NC, NS = 2, 16
NW = NC * NS
GCHUNK = 128
ROWS_PER_W = (B * K * P) // NW
MBLK = 2048


def _l2n(x):
    n = jnp.sqrt(jnp.sum(x * x, axis=1, keepdims=True))
    return x / jnp.maximum(n, 1e-12)


def _dotT(a, b):
    return lax.dot_general(a, b, dimension_numbers=(((1,), (1,)), ((), ())),
                           preferred_element_type=jnp.float32)


def _align_body(f_ref, t_ref, out_ref):
    p = pl.program_id(0)
    f = f_ref[0]
    t = t_ref[0]
    n1 = _l2n(f)
    t1 = _l2n(t)
    img_sim = _dotT(n1, n1) * (1.0 / TEMP)
    txt_sim = _dotT(t1, t1) * (1.0 / TEMP)
    mask = (img_sim > SIM_THR) & (txt_sim > SIM_THR)
    valid = jnp.any(mask, axis=1)
    vf = valid.astype(jnp.float32)
    col = jnp.where(valid[None, :], 0.0, -1e9)
    img_s = img_sim + col
    txt_s = txt_sim + col

    def lse(s):
        m = jnp.max(s, axis=-1, keepdims=True)
        return m + jnp.log(jnp.sum(jnp.exp(s - m), axis=-1, keepdims=True))

    img_lp = img_s - lse(img_s)
    txt_lp = txt_s - lse(txt_s)
    t_p = jnp.exp(txt_lp)
    i_p = jnp.exp(img_lp)
    w = vf[:, None] * vf[None, :]
    n = jnp.maximum(jnp.sum(vf), 1.0)
    kl1 = jnp.sum(w * t_p * (txt_lp - img_lp)) / n
    kl2 = jnp.sum(w * i_p * (img_lp - txt_lp)) / n
    lk = 0.5 * (kl1 + kl2)
    lk = jnp.where(jnp.sum(vf) > 0, lk, 0.0)
    prev = jnp.where(p == 0, 0.0, out_ref[0, 0])
    out_ref[...] = jnp.reshape(prev + lk, (1, 1))


def _align_call(feature, tft):
    return pl.pallas_call(
        _align_body,
        grid=(P,),
        in_specs=[
            pl.BlockSpec((1, B, D), lambda p: (p, 0, 0)),
            pl.BlockSpec((1, B, D), lambda p: (p, 0, 0)),
        ],
        out_specs=pl.BlockSpec((1, 1), lambda p: (0, 0)),
        out_shape=jax.ShapeDtypeStruct((1, 1), jnp.float32),
    )(feature, tft)


RB = 256


def _topk_body(t_ref, m_ref, out_ref):
    tf = t_ref[...]
    q = _l2n(jnp.mean(tf, axis=0))
    mn = _l2n(m_ref[...])
    s = _dotT(q, mn)
    iota = lax.broadcasted_iota(jnp.int32, (RB, M), 1)
    cols = []
    for _t in range(K):
        cur = jnp.max(s, axis=1)
        am = jnp.min(jnp.where(s == cur[:, None], iota, M), axis=1)
        cols.append(am)
        s = jnp.where(iota == am[:, None], -jnp.inf, s)
    out_ref[...] = jnp.stack(cols, axis=1)


def _topk_call(tft, mem_feats):
    return pl.pallas_call(
        _topk_body,
        grid=(B // RB,),
        in_specs=[
            pl.BlockSpec((P, RB, D), lambda i: (0, i, 0)),
            pl.BlockSpec((M, D), lambda i: (0, 0)),
        ],
        out_specs=pl.BlockSpec((RB, K), lambda i: (i, 0)),
        out_shape=jax.ShapeDtypeStruct((B, K), jnp.int32),
    )(tft, mem_feats)


def _sc_body(cflat, cross, posn, vid, posf, negm, pvout,
             idxsl, idx2, rows0, rows1, cfull, pvbuf, mask_v, vid_v, pos_v,
             gsem0, gsem1, osem0, osem1):
    wid = lax.axis_index("s") * NC + lax.axis_index("c")
    pp = wid // (NW // P)
    wb = (wid % (NW // P)) * ROWS_PER_W

    zeros16 = jnp.zeros((16,), jnp.float32)

    @pl.when(wid == 0)
    def _():
        pltpu.sync_copy(cross, cfull)
        pltpu.sync_copy(posn, pos_v)

        def init(i, _):
            mask_v[pl.ds(i * 16, 16)] = jnp.ones((16,), jnp.float32)
            return 0
        lax.fori_loop(0, M // 16, init, 0)

        def cscat(j, _):
            plsc.store_scatter(mask_v, [cfull[pl.ds(j * 16, 16)]], zeros16)
            return 0
        lax.fori_loop(0, (B * K) // 16, cscat, 0)

        def pscat(i, _):
            plsc.store_scatter(mask_v, [pos_v[pl.ds(i * 16, 16)]], zeros16)
            return 0
        lax.fori_loop(0, B // 16, pscat, 0)
        pltpu.sync_copy(mask_v, negm)

    @pl.when(wid == 1)
    def _():
        pltpu.sync_copy(cross, cfull)
        pltpu.sync_copy(vid, vid_v)

        def vg(j, _):
            pvbuf[pl.ds(j * 16, 16)] = plsc.load_gather(
                vid_v, [cfull[pl.ds(j * 16, 16)]])
            return 0
        lax.fori_loop(0, (B * K) // 16, vg, 0)
        pltpu.sync_copy(pvbuf, pvout)

    pltpu.sync_copy(cross.at[pl.ds(wb, ROWS_PER_W)], idxsl)

    def addp(i, _):
        idx2[pl.ds(i * 16, 16)] = idxsl[pl.ds(i * 16, 16)] + pp * M
        return 0
    lax.fori_loop(0, ROWS_PER_W // 16, addp, 0)

    nchunk = ROWS_PER_W // GCHUNK
    rows = (rows0, rows1)
    gsem = (gsem0, gsem1)
    osem = (osem0, osem1)
    obase = pp * (B * K) + wb
    for j in range(2):
        pltpu.async_copy(cflat.at[idx2.at[pl.ds(j * GCHUNK, GCHUNK)]],
                         rows[j], gsem[j])
    for j in range(nchunk):
        b = j % 2
        pltpu.make_async_copy(cflat.at[idx2.at[pl.ds(0, GCHUNK)]],
                              rows[b], gsem[b]).wait()
        pltpu.async_copy(rows[b], posf.at[pl.ds(obase + j * GCHUNK, GCHUNK)],
                         osem[b])
        if j + 2 < nchunk:
            pltpu.make_async_copy(rows[b],
                                  posf.at[pl.ds(obase, GCHUNK)],
                                  osem[b]).wait()
            pltpu.async_copy(
                cflat.at[idx2.at[pl.ds((j + 2) * GCHUNK, GCHUNK)]],
                rows[b], gsem[b])
    for j in range(2):
        pltpu.make_async_copy(rows[j], posf.at[pl.ds(obase, GCHUNK)],
                              osem[j]).wait()


@functools.lru_cache(maxsize=1)
def _make_sc_call():
    return functools.partial(
        pl.kernel,
        out_type=[
            jax.ShapeDtypeStruct((P * B * K, D), jnp.float32),
            jax.ShapeDtypeStruct((M,), jnp.float32),
            jax.ShapeDtypeStruct((B * K,), jnp.int32),
        ],
        mesh=plsc.VectorSubcoreMesh(core_axis_name="c", subcore_axis_name="s",
                                    num_cores=NC, num_subcores=NS),
        compiler_params=pltpu.CompilerParams(needs_layout_passes=False),
        scratch_types=[
            pltpu.VMEM((ROWS_PER_W,), jnp.int32),
            pltpu.VMEM((ROWS_PER_W,), jnp.int32),
            pltpu.VMEM((GCHUNK, D), jnp.float32),
            pltpu.VMEM((GCHUNK, D), jnp.float32),
            pltpu.VMEM((B * K,), jnp.int32),
            pltpu.VMEM((B * K,), jnp.int32),
            pltpu.VMEM((M,), jnp.float32),
            pltpu.VMEM((M,), jnp.int32),
            pltpu.VMEM((B,), jnp.int32),
            pltpu.SemaphoreType.DMA,
            pltpu.SemaphoreType.DMA,
            pltpu.SemaphoreType.DMA,
            pltpu.SemaphoreType.DMA,
        ],
    )(_sc_body)


def _sc_call(cflat, crossf, posn, vid):
    return _make_sc_call()(cflat, crossf, posn, vid)


def _y_body(f_ref, c_ref, nm_ref, out_ref):
    mb = pl.program_id(1)
    f = f_ref[0]
    c = c_ref[0]
    nm = nm_ref[0]
    a2 = jnp.sum(f * f, axis=1) + 1e-12
    b2 = jnp.sum(c * c, axis=1)
    g = lax.dot_general((-2.0 * f).astype(jnp.bfloat16),
                        c.astype(jnp.bfloat16),
                        dimension_numbers=(((1,), (1,)), ((), ())),
                        preferred_element_type=jnp.float32)
    d2 = jnp.maximum(a2[:, None] + (g + b2[None, :]), 1e-12)
    e = jnp.exp(-SCALE * jnp.sqrt(d2))
    part = lax.dot_general(e, nm, dimension_numbers=(((1,), (0,)), ((), ())),
                           preferred_element_type=jnp.float32)
    prev = jnp.where(mb == 0, jnp.zeros_like(part), out_ref[0, 0])
    out_ref[...] = (prev + part)[None, None, :]


def _y_call(feature, centers, negm2d):
    return pl.pallas_call(
        _y_body,
        grid=(P, M // MBLK),
        in_specs=[
            pl.BlockSpec((1, B, D), lambda p, mb: (p, 0, 0)),
            pl.BlockSpec((1, MBLK, D), lambda p, mb: (p, mb, 0)),
            pl.BlockSpec((1, MBLK), lambda p, mb: (0, mb)),
        ],
        out_specs=pl.BlockSpec((1, 1, B), lambda p, mb: (p, 0, 0)),
        out_shape=jax.ShapeDtypeStruct((P, 1, B), jnp.float32),
    )(feature, centers, negm2d)


def _final_body(pf_ref, f_ref, ys_ref, al_ref, tot_ref, con_ref, acc_ref):
    p = pl.program_id(0)
    pf = pf_ref[0].reshape(B, K, D)
    f = f_ref[0]
    diff = f[:, None, :] - pf
    d2 = jnp.sum(diff * diff, axis=-1) + 1e-12
    x = jnp.log(jnp.sum(jnp.exp(-SCALE * jnp.sqrt(d2)), axis=1))
    y = jnp.log(ys_ref[0, 0])
    l = jnp.sum(y - x) / B
    l = jnp.where(jnp.isnan(l), 0.0, l)
    prev = jnp.where(p == 0, 0.0, acc_ref[0, 0])
    tot = prev + l
    acc_ref[...] = jnp.reshape(tot, (1, 1))

    @pl.when(p == P - 1)
    def _():
        con = tot / P
        con_ref[...] = jnp.reshape(con, (1, 1))
        tot_ref[...] = jnp.reshape(con + KLW * al_ref[0, 0], (1, 1))


def _final_call(posf3, feature, ysum, align):
    return pl.pallas_call(
        _final_body,
        grid=(P,),
        in_specs=[
            pl.BlockSpec((1, B * K, D), lambda p: (p, 0, 0)),
            pl.BlockSpec((1, B, D), lambda p: (p, 0, 0)),
            pl.BlockSpec((1, 1, B), lambda p: (p, 0, 0)),
            pl.BlockSpec((1, 1), lambda p: (0, 0)),
        ],
        out_specs=[
            pl.BlockSpec((1, 1), lambda p: (0, 0)),
            pl.BlockSpec((1, 1), lambda p: (0, 0)),
        ],
        out_shape=[
            jax.ShapeDtypeStruct((1, 1), jnp.float32),
            jax.ShapeDtypeStruct((1, 1), jnp.float32),
        ],
        scratch_shapes=[pltpu.VMEM((1, 1), jnp.float32)],
    )(posf3, feature, ysum, align)


def kernel(feature, text_feature, centers, text_centers, position,
           mem_feats, vid, domains):
    tft = jnp.transpose(text_feature, (1, 0, 2))
    align = _align_call(feature, tft)
    cross = _topk_call(tft, mem_feats)
    posf, negm, pvf = _sc_call(
        centers.reshape(P * M, D), cross.reshape(-1),
        position.astype(jnp.int32), vid)
    ysum = _y_call(feature, centers, negm.reshape(1, M))
    total, contr = _final_call(posf.reshape(P, B * K, D), feature, ysum, align)
    return (total.reshape(()), contr.reshape(()), align.reshape(()),
            pvf.reshape(B, K))
